# Initial kernel scaffold; baseline (speedup 1.0000x reference)
import jax
import jax.numpy as jnp
from jax.experimental import pallas as pl


def kernel(x, token_table, pos_table):
    raise NotImplementedError("write your pallas kernel here")


# speedup vs baseline: 2.7176x; 2.7176x over previous
import functools

import jax
import jax.numpy as jnp
from jax import lax
from jax.experimental import pallas as pl
from jax.experimental.pallas import tpu as pltpu
from jax.experimental.pallas import tpu_sc as plsc

VOCAB = 100000
CTX = 200
DIM = 64
BATCH = 1024

NC = 2
NS = 16
NW = NC * NS
ROWS = BATCH * CTX
RPW = ROWS // NW
SEQS_PER_W = RPW // CTX


def _sc_body(x_hbm, tok_hbm, pos_hbm, out_hbm, idx_v, buf, posb, sem):
  wid = lax.axis_index("s") * NC + lax.axis_index("c")
  base = wid * RPW

  pltpu.sync_copy(x_hbm.at[pl.ds(base, RPW)], idx_v)
  pltpu.sync_copy(pos_hbm, posb)

  @pl.loop(0, SEQS_PER_W)
  def _chunk(g):
    c0 = g * CTX
    d1 = pltpu.async_copy(
        tok_hbm.at[idx_v.at[pl.ds(c0, 128)]], buf.at[pl.ds(0, 128)], sem)
    d2 = pltpu.async_copy(
        tok_hbm.at[idx_v.at[pl.ds(c0 + 128, CTX - 128)]],
        buf.at[pl.ds(128, CTX - 128)], sem)
    d1.wait()
    d2.wait()

    @pl.loop(0, CTX)
    def _row(r):
      for c in range(DIM // 16):
        sl = pl.ds(c * 16, 16)
        buf[r, sl] = buf[r, sl] + posb[r, sl]

    pltpu.sync_copy(buf, out_hbm.at[pl.ds(base + c0, CTX)])


@jax.jit
def _sc_embed(x_flat, token_table, pos_table):
  mesh = plsc.VectorSubcoreMesh(core_axis_name="c", subcore_axis_name="s")
  return pl.kernel(
      _sc_body,
      out_type=jax.ShapeDtypeStruct((ROWS, DIM), jnp.float32),
      mesh=mesh,
      scratch_types=[
          pltpu.VMEM((RPW,), jnp.int32),
          pltpu.VMEM((CTX, DIM), jnp.float32),
          pltpu.VMEM((CTX, DIM), jnp.float32),
          pltpu.SemaphoreType.DMA,
      ],
      compiler_params=pltpu.CompilerParams(use_tc_tiling_on_sc=False),
  )(x_flat, token_table, pos_table)


def kernel(x, token_table, pos_table):
  x_flat = x.reshape(-1).astype(jnp.int32)
  out = _sc_embed(x_flat, token_table, pos_table)
  return out.reshape(BATCH, CTX, DIM)


# speedup vs baseline: 3.1250x; 1.1499x over previous
import functools

import jax
import jax.numpy as jnp
from jax import lax
from jax.experimental import pallas as pl
from jax.experimental.pallas import tpu as pltpu
from jax.experimental.pallas import tpu_sc as plsc

VOCAB = 100000
CTX = 200
DIM = 64
BATCH = 1024

NC = 2
NS = 16
NW = NC * NS
ROWS = BATCH * CTX
RPW = ROWS // NW
SEQS_PER_W = RPW // CTX


def _sc_body(x_hbm, tok_hbm, pos_hbm, out_hbm, idx_v, buf0, buf1, posb, sem):
  wid = lax.axis_index("s") * NC + lax.axis_index("c")
  base = wid * RPW

  pltpu.sync_copy(x_hbm.at[pl.ds(base, RPW)], idx_v)

  @pl.when(lax.axis_index("s") == 0)
  def _stage_pos():
    pltpu.sync_copy(pos_hbm, posb)

  plsc.subcore_barrier()

  def fire(g, buf):
    c0 = g * CTX
    pltpu.sync_copy(posb, buf)
    d1 = pltpu.async_copy(
        tok_hbm.at[idx_v.at[pl.ds(c0, 128)]], buf.at[pl.ds(0, 128)], sem,
        add=True)
    d2 = pltpu.async_copy(
        tok_hbm.at[idx_v.at[pl.ds(c0 + 128, CTX - 128)]],
        buf.at[pl.ds(128, CTX - 128)], sem, add=True)
    return d1, d2

  def drain(g, buf, descs):
    for d in descs:
      d.wait()
    pltpu.sync_copy(buf, out_hbm.at[pl.ds(base + g * CTX, CTX)])

  @pl.loop(0, SEQS_PER_W, step=2)
  def _pair(g):
    dA = fire(g, buf0)
    dB = fire(g + 1, buf1)
    drain(g, buf0, dA)
    drain(g + 1, buf1, dB)


@jax.jit
def _sc_embed(x_flat, token_table, pos_table):
  mesh = plsc.VectorSubcoreMesh(core_axis_name="c", subcore_axis_name="s")
  return pl.kernel(
      _sc_body,
      out_type=jax.ShapeDtypeStruct((ROWS, DIM), jnp.float32),
      mesh=mesh,
      scratch_types=[
          pltpu.VMEM((RPW,), jnp.int32),
          pltpu.VMEM((CTX, DIM), jnp.float32),
          pltpu.VMEM((CTX, DIM), jnp.float32),
          pltpu.VMEM_SHARED((CTX, DIM), jnp.float32),
          pltpu.SemaphoreType.DMA,
      ],
      compiler_params=pltpu.CompilerParams(use_tc_tiling_on_sc=False),
  )(x_flat, token_table, pos_table)


def kernel(x, token_table, pos_table):
  x_flat = x.reshape(-1).astype(jnp.int32)
  out = _sc_embed(x_flat, token_table, pos_table)
  return out.reshape(BATCH, CTX, DIM)
